# Initial kernel scaffold; baseline (speedup 1.0000x reference)
import jax
import jax.numpy as jnp
from jax.experimental import pallas as pl


def kernel(x, Wg, bg, We, be):
    raise NotImplementedError("write your pallas kernel here")


# speedup vs baseline: 2.5054x; 2.5054x over previous
import functools

import jax
import jax.numpy as jnp
from jax.experimental import pallas as pl
from jax.experimental.pallas import tpu as pltpu

H = 1024
E = 8
TOPK = 2
EPS = 1e-06


def _moe_body(x_ref, wg_ref, bg_ref, we_ref, be_ref,
              out_ref, aux_ref, w1_ref, w2_ref, i1_ref, i2_ref, probs_sum_ref):
    e = pl.program_id(0)
    n = x_ref.shape[0]

    @pl.when(e == 0)
    def _router():
        logits = jax.lax.dot_general(
            x_ref[...], wg_ref[...], (((1,), (1,)), ((), ())),
            precision=jax.lax.Precision.DEFAULT,
            preferred_element_type=jnp.float32) + bg_ref[...][None, :]
        m = jnp.max(logits, axis=1, keepdims=True)
        ex = jnp.exp(logits - m)
        probs = ex / jnp.sum(ex, axis=1, keepdims=True)
        iota = jax.lax.broadcasted_iota(jnp.int32, (n, E), 1)
        p1 = jnp.max(probs, axis=1, keepdims=True)
        i1 = jnp.min(jnp.where(probs == p1, iota, E), axis=1, keepdims=True)
        masked = jnp.where(iota == i1, -jnp.inf, probs)
        p2 = jnp.max(masked, axis=1, keepdims=True)
        i2 = jnp.min(jnp.where(masked == p2, iota, E), axis=1, keepdims=True)
        denom = p1 + p2 + EPS
        w1_ref[...] = p1 / denom
        w2_ref[...] = p2 / denom
        i1_ref[...] = i1
        i2_ref[...] = i2
        mask = ((iota == i1) | (iota == i2)).astype(jnp.float32)
        usage = jnp.mean(mask, axis=0)
        gates = jnp.mean(probs, axis=0)
        aux_ref[0, 0] = jnp.sum(usage * gates) * E
        probs_sum_ref[...] = jnp.zeros_like(probs_sum_ref)

    w_col = (jnp.where(i1_ref[...] == e, w1_ref[...], 0.0)
             + jnp.where(i2_ref[...] == e, w2_ref[...], 0.0))

    xb = x_ref[...].astype(jnp.bfloat16)
    web = we_ref[0].astype(jnp.bfloat16)
    y = jax.lax.dot_general(
        xb, web, (((1,), (1,)), ((), ())),
        preferred_element_type=jnp.float32) + be_ref[0]
    contrib = w_col * y

    @pl.when(e == 0)
    def _init():
        out_ref[...] = contrib

    @pl.when(e > 0)
    def _acc():
        out_ref[...] += contrib


@jax.jit
def kernel(x, Wg, bg, We, be):
    b, s, h = x.shape
    x_flat = x.reshape(-1, h)
    n = x_flat.shape[0]

    out, aux = pl.pallas_call(
        _moe_body,
        grid=(E,),
        in_specs=[
            pl.BlockSpec((n, h), lambda e: (0, 0)),
            pl.BlockSpec((E, h), lambda e: (0, 0)),
            pl.BlockSpec((E,), lambda e: (0,)),
            pl.BlockSpec((1, h, h), lambda e: (e, 0, 0)),
            pl.BlockSpec((1, 1, h), lambda e: (e, 0, 0)),
        ],
        out_specs=[
            pl.BlockSpec((n, h), lambda e: (0, 0)),
            pl.BlockSpec(memory_space=pltpu.SMEM),
        ],
        out_shape=[
            jax.ShapeDtypeStruct((n, h), jnp.float32),
            jax.ShapeDtypeStruct((1, 1), jnp.float32),
        ],
        scratch_shapes=[
            pltpu.VMEM((n, 1), jnp.float32),
            pltpu.VMEM((n, 1), jnp.float32),
            pltpu.VMEM((n, 1), jnp.int32),
            pltpu.VMEM((n, 1), jnp.int32),
            pltpu.VMEM((n, E), jnp.float32),
        ],
    )(x_flat, Wg, bg, We, be.reshape(E, 1, h))

    return out.reshape(b, s, h), aux[0, 0]
